# baseline (speedup 1.0000x reference)
import functools

import jax
import jax.numpy as jnp
from jax import lax
from jax.experimental import pallas as pl
from jax.experimental.pallas import tpu as pltpu
from jax.experimental.pallas import tpu_sc as plsc

EMBED_DIM = 64
BATCH = 16384
N_FIELDS = 26
TOTAL = BATCH * N_FIELDS

NUM_CORES = 2
NUM_SUBCORES = 16
NUM_WORKERS = NUM_CORES * NUM_SUBCORES
ROWS_PER_WORKER = TOTAL // NUM_WORKERS
CHUNK = 1024
N_CHUNKS = ROWS_PER_WORKER // CHUNK


def _gather_body(table_hbm, idx_hbm, out_hbm, idx_v, rows_v, sem):
    wid = lax.axis_index("s") * NUM_CORES + lax.axis_index("c")
    base = wid * ROWS_PER_WORKER

    def step(i, carry):
        off = base + i * CHUNK
        pltpu.sync_copy(idx_hbm.at[pl.ds(off, CHUNK)], idx_v)
        pltpu.async_copy(table_hbm.at[idx_v], rows_v, sem).wait()
        pltpu.sync_copy(rows_v, out_hbm.at[pl.ds(off, CHUNK)])
        return carry

    lax.fori_loop(0, N_CHUNKS, step, 0)


def kernel(input_indices, W):
    idx = input_indices.reshape(-1).astype(jnp.int32)
    mesh = plsc.VectorSubcoreMesh(core_axis_name="c", subcore_axis_name="s")
    out = pl.kernel(
        _gather_body,
        out_type=jax.ShapeDtypeStruct((TOTAL, EMBED_DIM), jnp.float32),
        mesh=mesh,
        scratch_types=[
            pltpu.VMEM((CHUNK,), jnp.int32),
            pltpu.VMEM((CHUNK, EMBED_DIM), jnp.float32),
            pltpu.SemaphoreType.DMA,
        ],
        compiler_params=pltpu.CompilerParams(use_tc_tiling_on_sc=False),
    )(W, idx)
    return out.reshape(BATCH, N_FIELDS, EMBED_DIM)


# speedup vs baseline: 1.0133x; 1.0133x over previous
import jax
import jax.numpy as jnp
from jax import lax
from jax.experimental import pallas as pl
from jax.experimental.pallas import tpu as pltpu
from jax.experimental.pallas import tpu_sc as plsc

EMBED_DIM = 64
BATCH = 16384
N_FIELDS = 26
TOTAL = BATCH * N_FIELDS

NUM_CORES = 2
NUM_SUBCORES = 16
NUM_WORKERS = NUM_CORES * NUM_SUBCORES
ROWS_PER_WORKER = TOTAL // NUM_WORKERS
CHUNK = 832
N_CHUNKS = ROWS_PER_WORKER // CHUNK


def _gather_body(table_hbm, idx_hbm, out_hbm,
                 idx_all, rows0, rows1, g0, g1, s0, s1):
    wid = lax.axis_index("s") * NUM_CORES + lax.axis_index("c")
    base = wid * ROWS_PER_WORKER
    pltpu.sync_copy(idx_hbm.at[pl.ds(base, ROWS_PER_WORKER)], idx_all)

    rows = (rows0, rows1)
    gsem = (g0, g1)
    ssem = (s0, s1)
    gathers = [None, None]
    stores = [None, None]
    for i in range(N_CHUNKS + 1):
        b = i % 2
        if i < N_CHUNKS:
            if stores[b] is not None:
                stores[b].wait()
            gathers[b] = pltpu.async_copy(
                table_hbm.at[idx_all.at[pl.ds(i * CHUNK, CHUNK)]],
                rows[b], gsem[b])
        if i >= 1:
            pb = (i - 1) % 2
            gathers[pb].wait()
            stores[pb] = pltpu.async_copy(
                rows[pb], out_hbm.at[pl.ds(base + (i - 1) * CHUNK, CHUNK)],
                ssem[pb])
    stores[0].wait()
    stores[1].wait()


def kernel(input_indices, W):
    idx = input_indices.reshape(-1).astype(jnp.int32)
    mesh = plsc.VectorSubcoreMesh(core_axis_name="c", subcore_axis_name="s")
    out = pl.kernel(
        _gather_body,
        out_type=jax.ShapeDtypeStruct((TOTAL, EMBED_DIM), jnp.float32),
        mesh=mesh,
        scratch_types=[
            pltpu.VMEM((ROWS_PER_WORKER,), jnp.int32),
            pltpu.VMEM((CHUNK, EMBED_DIM), jnp.float32),
            pltpu.VMEM((CHUNK, EMBED_DIM), jnp.float32),
            pltpu.SemaphoreType.DMA,
            pltpu.SemaphoreType.DMA,
            pltpu.SemaphoreType.DMA,
            pltpu.SemaphoreType.DMA,
        ],
        compiler_params=pltpu.CompilerParams(use_tc_tiling_on_sc=False),
    )(W, idx)
    return out.reshape(BATCH, N_FIELDS, EMBED_DIM)
